# Initial kernel scaffold; baseline (speedup 1.0000x reference)
import jax
import jax.numpy as jnp
from jax.experimental import pallas as pl


def kernel(x, table):
    raise NotImplementedError("write your pallas kernel here")


# speedup vs baseline: 45.2935x; 45.2935x over previous
import functools

import jax
import jax.numpy as jnp
import numpy as np
from jax import lax
from jax.experimental import pallas as pl
from jax.experimental.pallas import tpu as pltpu
from jax.experimental.pallas import tpu_sc as plsc

DIM = 3
NFEAT = 8
HASHMAP = 524288
RES = 128.0
N = 524288

NC, NS, L = 2, 16, 16
NW = NC * NS
PPW = N // NW
C = 512
NG = C // L
NCHUNK = PPW // C

P1 = np.int32(np.uint32(2654435761).astype(np.int32))
P2 = np.int32(805459861)
HMASK = np.int32(HASHMAP - 1)


def kernel(x, table):
    mesh = plsc.VectorSubcoreMesh(core_axis_name="c", subcore_axis_name="s")

    @functools.partial(
        pl.kernel,
        mesh=mesh,
        out_type=jax.ShapeDtypeStruct((N, NFEAT), jnp.float32),
        scratch_types=[
            pltpu.VMEM((C, DIM), jnp.float32),
            pltpu.VMEM((NG, 8 * L), jnp.int32),
            pltpu.VMEM((C * 8, NFEAT), jnp.float32),
            pltpu.VMEM((C, NFEAT), jnp.float32),
            pltpu.SemaphoreType.DMA,
        ],
        compiler_params=pltpu.CompilerParams(
            needs_layout_passes=False, use_tc_tiling_on_sc=False
        ),
    )
    def grid_lookup(x_hbm, table_hbm, out_hbm, xbuf, idxbuf, rowsbuf, outbuf, sem):
        i32 = jnp.int32
        wid = (lax.axis_index("s").astype(i32) * i32(NC)
               + lax.axis_index("c").astype(i32))
        iota = lax.iota(jnp.int32, L)
        c0 = jnp.full((L,), 0, jnp.int32)
        c1 = jnp.full((L,), 1, jnp.int32)
        c2 = jnp.full((L,), 2, jnp.int32)
        fcols = [jnp.full((L,), f, jnp.int32) for f in range(NFEAT)]

        def load_xs(g):
            ridx = g * i32(L) + iota
            xs0 = plsc.load_gather(xbuf, [ridx, c0]) * RES
            xs1 = plsc.load_gather(xbuf, [ridx, c1]) * RES
            xs2 = plsc.load_gather(xbuf, [ridx, c2]) * RES
            return xs0, xs1, xs2

        def hash_group(g, _):
            xs0, xs1, xs2 = load_xs(g)
            xi0 = xs0.astype(jnp.int32)
            xi1 = xs1.astype(jnp.int32)
            xi2 = xs2.astype(jnp.int32)
            h0, hp0 = xi0, xi0 + i32(1)
            h1 = xi1 * P1
            hp1 = h1 + P1
            h2 = xi2 * P2
            hp2 = h2 + P2
            for c in range(8):
                a = hp0 if c & 1 else h0
                b = hp1 if c & 2 else h1
                d = hp2 if c & 4 else h2
                idxbuf[g, pl.ds(c * L, L)] = (a ^ b ^ d) & HMASK
            return i32(0)

        def compute_group(g, _):
            ridx = g * i32(L) + iota
            xs0, xs1, xs2 = load_xs(g)
            xf0 = xs0 - xs0.astype(jnp.int32).astype(jnp.float32)
            xf1 = xs1 - xs1.astype(jnp.int32).astype(jnp.float32)
            xf2 = xs2 - xs2.astype(jnp.int32).astype(jnp.float32)
            m0, m1, m2 = 1.0 - xf0, 1.0 - xf1, 1.0 - xf2
            ws, rbases = [], []
            for c in range(8):
                t0 = xf0 if c & 1 else m0
                t1 = xf1 if c & 2 else m1
                t2 = xf2 if c & 4 else m2
                ws.append(t0 * t1 * t2)
                rbases.append(g * i32(8 * L) + i32(c * L) + iota)
            for f in range(NFEAT):
                acc = ws[0] * plsc.load_gather(rowsbuf, [rbases[0], fcols[f]])
                for c in range(1, 8):
                    acc = acc + ws[c] * plsc.load_gather(rowsbuf, [rbases[c], fcols[f]])
                plsc.store_scatter(outbuf, [ridx, fcols[f]], acc)
            return i32(0)

        def chunk_body(k, _):
            cbase = wid * i32(PPW) + k * i32(C)
            pltpu.sync_copy(x_hbm.at[pl.ds(cbase, C)], xbuf)
            lax.fori_loop(i32(0), i32(NG), hash_group, i32(0))
            copies = [
                pltpu.async_copy(
                    table_hbm.at[idxbuf.at[np.int32(g)]],
                    rowsbuf.at[pl.ds(g * (8 * L), 8 * L)],
                    sem,
                )
                for g in range(NG)
            ]
            for cp in copies:
                cp.wait()
            lax.fori_loop(i32(0), i32(NG), compute_group, i32(0))
            pltpu.sync_copy(outbuf, out_hbm.at[pl.ds(cbase, C)])
            return i32(0)

        lax.fori_loop(i32(0), i32(NCHUNK), chunk_body, i32(0))

    return grid_lookup(x, table)


# speedup vs baseline: 55.6085x; 1.2277x over previous
import functools

import jax
import jax.numpy as jnp
import numpy as np
from jax import lax
from jax.experimental import pallas as pl
from jax.experimental.pallas import tpu as pltpu
from jax.experimental.pallas import tpu_sc as plsc

DIM = 3
NFEAT = 8
HASHMAP = 524288
RES = 128.0
N = 524288

NC, NS, L = 2, 16, 16
NW = NC * NS
PPW = N // NW
C = 512
NG = C // L
NCHUNK = PPW // C

P1 = np.int32(np.uint32(2654435761).astype(np.int32))
P2 = np.int32(805459861)
HMASK = np.int32(HASHMAP - 1)


def kernel(x, table):
    mesh = plsc.VectorSubcoreMesh(core_axis_name="c", subcore_axis_name="s")

    @functools.partial(
        pl.kernel,
        mesh=mesh,
        out_type=jax.ShapeDtypeStruct((N, NFEAT), jnp.float32),
        scratch_types=[
            pltpu.VMEM((2, C * DIM), jnp.float32),
            pltpu.VMEM((2 * NG, 8 * L), jnp.int32),
            pltpu.VMEM((2, C * 8, NFEAT), jnp.float32),
            pltpu.VMEM((C, NFEAT), jnp.float32),
            pltpu.SemaphoreType.DMA,
            pltpu.SemaphoreType.DMA,
        ],
        compiler_params=pltpu.CompilerParams(
            needs_layout_passes=False, use_tc_tiling_on_sc=False
        ),
    )
    def grid_lookup(x_hbm, table_hbm, out_hbm, xbuf, idxbuf, rowsbuf, outbuf,
                    sem0, sem1):
        i32 = jnp.int32
        wid = (lax.axis_index("s").astype(i32) * i32(NC)
               + lax.axis_index("c").astype(i32))
        iota = lax.iota(jnp.int32, L)
        fcols = [jnp.full((L,), f, jnp.int32) for f in range(NFEAT)]
        sems = (sem0, sem1)

        def load_xs(p, g):
            idx3 = (g * i32(L) + iota) * i32(DIM)
            xb = xbuf.at[np.int32(p)]
            xs0 = plsc.load_gather(xb, [idx3]) * RES
            xs1 = plsc.load_gather(xb, [idx3 + i32(1)]) * RES
            xs2 = plsc.load_gather(xb, [idx3 + i32(2)]) * RES
            return xs0, xs1, xs2

        def make_hash_group(p):
            def hash_group(g, _):
                xs0, xs1, xs2 = load_xs(p, g)
                xi0 = xs0.astype(jnp.int32)
                xi1 = xs1.astype(jnp.int32)
                xi2 = xs2.astype(jnp.int32)
                h0, hp0 = xi0, xi0 + i32(1)
                h1 = xi1 * P1
                hp1 = h1 + P1
                h2 = xi2 * P2
                hp2 = h2 + P2
                for c in range(8):
                    a = hp0 if c & 1 else h0
                    b = hp1 if c & 2 else h1
                    d = hp2 if c & 4 else h2
                    idxbuf[i32(p * NG) + g, pl.ds(c * L, L)] = (a ^ b ^ d) & HMASK
                return i32(0)

            return hash_group

        hash_groups = [make_hash_group(0), make_hash_group(1)]

        def make_compute_group(p):
            def compute_group(g, _):
                ridx = g * i32(L) + iota
                xs0, xs1, xs2 = load_xs(p, g)
                rb = rowsbuf.at[np.int32(p)]
                xf0 = xs0 - xs0.astype(jnp.int32).astype(jnp.float32)
                xf1 = xs1 - xs1.astype(jnp.int32).astype(jnp.float32)
                xf2 = xs2 - xs2.astype(jnp.int32).astype(jnp.float32)
                m0, m1, m2 = 1.0 - xf0, 1.0 - xf1, 1.0 - xf2
                ws, rbases = [], []
                for c in range(8):
                    t0 = xf0 if c & 1 else m0
                    t1 = xf1 if c & 2 else m1
                    t2 = xf2 if c & 4 else m2
                    ws.append(t0 * t1 * t2)
                    rbases.append(g * i32(8 * L) + i32(c * L) + iota)
                for f in range(NFEAT):
                    acc = ws[0] * plsc.load_gather(rb, [rbases[0], fcols[f]])
                    for c in range(1, 8):
                        acc = acc + ws[c] * plsc.load_gather(rb, [rbases[c], fcols[f]])
                    plsc.store_scatter(outbuf, [ridx, fcols[f]], acc)
                return i32(0)

            return compute_group

        compute_groups = [make_compute_group(0), make_compute_group(1)]

        def cbase_of(k):
            return wid * i32(PPW) + k * i32(C)

        def prepare(p, k):
            pltpu.sync_copy(x_hbm.at[pl.ds(cbase_of(k) * i32(DIM), C * DIM)],
                            xbuf.at[np.int32(p)])
            lax.fori_loop(i32(0), i32(NG), hash_groups[p], i32(0))
            for g in range(NG):
                pltpu.async_copy(
                    table_hbm.at[idxbuf.at[np.int32(p * NG + g)]],
                    rowsbuf.at[np.int32(p), pl.ds(g * (8 * L), 8 * L)],
                    sems[p],
                )

        def finish(p, k):
            for g in range(NG):
                pltpu.make_async_copy(
                    table_hbm.at[idxbuf.at[np.int32(p * NG + g)]],
                    rowsbuf.at[np.int32(p), pl.ds(g * (8 * L), 8 * L)],
                    sems[p],
                ).wait()
            lax.fori_loop(i32(0), i32(NG), compute_groups[p], i32(0))
            pltpu.sync_copy(outbuf, out_hbm.at[pl.ds(cbase_of(k), C)])

        prepare(0, i32(0))

        def pair_body(k2, _):
            k0 = k2 * i32(2)
            prepare(1, k0 + i32(1))
            finish(0, k0)

            @pl.when(k0 + i32(2) < i32(NCHUNK))
            def _():
                prepare(0, k0 + i32(2))

            finish(1, k0 + i32(1))
            return i32(0)

        lax.fori_loop(i32(0), i32(NCHUNK // 2), pair_body, i32(0))

    return grid_lookup(jnp.reshape(x, (N * DIM,)), table)


# speedup vs baseline: 60.1165x; 1.0811x over previous
import functools

import jax
import jax.numpy as jnp
import numpy as np
from jax import lax
from jax.experimental import pallas as pl
from jax.experimental.pallas import tpu as pltpu
from jax.experimental.pallas import tpu_sc as plsc

DIM = 3
NFEAT = 8
HASHMAP = 524288
RES = 128.0
N = 524288

NC, NS, L = 2, 16, 16
NW = NC * NS
PPW = N // NW
C = 512
NG = C // L
NCHUNK = PPW // C

P1 = np.int32(np.uint32(2654435761).astype(np.int32))
P2 = np.int32(805459861)
HMASK = np.int32(HASHMAP - 1)


def kernel(x, table):
    mesh = plsc.VectorSubcoreMesh(core_axis_name="c", subcore_axis_name="s")

    @functools.partial(
        pl.kernel,
        mesh=mesh,
        out_type=jax.ShapeDtypeStruct((N * NFEAT,), jnp.float32),
        scratch_types=[
            pltpu.VMEM((2, C * DIM), jnp.float32),
            pltpu.VMEM((2 * NG, 8 * L), jnp.int32),
            pltpu.VMEM((2, C * 8, NFEAT), jnp.float32),
            pltpu.VMEM((C * NFEAT,), jnp.float32),
            pltpu.SemaphoreType.DMA,
            pltpu.SemaphoreType.DMA,
        ],
        compiler_params=pltpu.CompilerParams(
            needs_layout_passes=False, use_tc_tiling_on_sc=False
        ),
    )
    def grid_lookup(x_hbm, table_hbm, out_hbm, xbuf, idxbuf, rowsbuf, outbuf,
                    sem0, sem1):
        i32 = jnp.int32
        wid = (lax.axis_index("s").astype(i32) * i32(NC)
               + lax.axis_index("c").astype(i32))
        iota = lax.iota(jnp.int32, L)
        sems = (sem0, sem1)
        half = iota // i32(8)
        pidx = [jnp.full((L,), 2 * k, jnp.int32) + half for k in range(8)]
        fcol2 = iota % i32(8)

        _dn = lax.GatherDimensionNumbers(
            offset_dims=(), collapsed_slice_dims=(0,), start_index_map=(0,)
        )

        def vgather(v, idx16):
            return lax.gather(
                v, idx16[:, None], _dn, (1,),
                mode=lax.GatherScatterMode.PROMISE_IN_BOUNDS,
            )

        def load_xs(p, g):
            idx3 = (g * i32(L) + iota) * i32(DIM)
            xb = xbuf.at[np.int32(p)]
            xs0 = plsc.load_gather(xb, [idx3]) * RES
            xs1 = plsc.load_gather(xb, [idx3 + i32(1)]) * RES
            xs2 = plsc.load_gather(xb, [idx3 + i32(2)]) * RES
            return xs0, xs1, xs2

        def make_hash_group(p):
            def hash_group(g, _):
                xs0, xs1, xs2 = load_xs(p, g)
                xi0 = xs0.astype(jnp.int32)
                xi1 = xs1.astype(jnp.int32)
                xi2 = xs2.astype(jnp.int32)
                h0, hp0 = xi0, xi0 + i32(1)
                h1 = xi1 * P1
                hp1 = h1 + P1
                h2 = xi2 * P2
                hp2 = h2 + P2
                for c in range(8):
                    a = hp0 if c & 1 else h0
                    b = hp1 if c & 2 else h1
                    d = hp2 if c & 4 else h2
                    idxbuf[i32(p * NG) + g, pl.ds(c * L, L)] = (a ^ b ^ d) & HMASK
                return i32(0)

            return hash_group

        hash_groups = [make_hash_group(0), make_hash_group(1)]

        def make_compute_group(p):
            def compute_group(g, _):
                xs0, xs1, xs2 = load_xs(p, g)
                rb = rowsbuf.at[np.int32(p)]
                xf0 = xs0 - xs0.astype(jnp.int32).astype(jnp.float32)
                xf1 = xs1 - xs1.astype(jnp.int32).astype(jnp.float32)
                xf2 = xs2 - xs2.astype(jnp.int32).astype(jnp.float32)
                m0, m1, m2 = 1.0 - xf0, 1.0 - xf1, 1.0 - xf2
                ws = []
                for c in range(8):
                    t0 = xf0 if c & 1 else m0
                    t1 = xf1 if c & 2 else m1
                    t2 = xf2 if c & 4 else m2
                    ws.append(t0 * t1 * t2)
                acc = [None] * 8
                for c in range(8):
                    rbase = g * i32(8 * L) + i32(c * L)
                    wc = ws[c]
                    for k in range(8):
                        v = plsc.load_gather(rb, [rbase + pidx[k], fcol2])
                        wp = vgather(wc, pidx[k])
                        acc[k] = wp * v if c == 0 else acc[k] + wp * v
                gout = g * i32(L * NFEAT)
                for k in range(8):
                    outbuf[pl.ds(gout + i32(k * L), L)] = acc[k]
                return i32(0)

            return compute_group

        compute_groups = [make_compute_group(0), make_compute_group(1)]

        def cbase_of(k):
            return wid * i32(PPW) + k * i32(C)

        def prepare(p, k):
            pltpu.sync_copy(x_hbm.at[pl.ds(cbase_of(k) * i32(DIM), C * DIM)],
                            xbuf.at[np.int32(p)])
            lax.fori_loop(i32(0), i32(NG), hash_groups[p], i32(0))
            for g in range(NG):
                pltpu.async_copy(
                    table_hbm.at[idxbuf.at[np.int32(p * NG + g)]],
                    rowsbuf.at[np.int32(p), pl.ds(g * (8 * L), 8 * L)],
                    sems[p],
                )

        def finish(p, k):
            for g in range(NG):
                pltpu.make_async_copy(
                    table_hbm.at[idxbuf.at[np.int32(p * NG + g)]],
                    rowsbuf.at[np.int32(p), pl.ds(g * (8 * L), 8 * L)],
                    sems[p],
                ).wait()
            lax.fori_loop(i32(0), i32(NG), compute_groups[p], i32(0))
            pltpu.sync_copy(outbuf,
                            out_hbm.at[pl.ds(cbase_of(k) * i32(NFEAT), C * NFEAT)])

        prepare(0, i32(0))

        def pair_body(k2, _):
            k0 = k2 * i32(2)
            prepare(1, k0 + i32(1))
            finish(0, k0)

            @pl.when(k0 + i32(2) < i32(NCHUNK))
            def _():
                prepare(0, k0 + i32(2))

            finish(1, k0 + i32(1))
            return i32(0)

        lax.fori_loop(i32(0), i32(NCHUNK // 2), pair_body, i32(0))

    table_lin = lax.optimization_barrier(jnp.reshape(table, (HASHMAP * NFEAT,)))
    out_flat = grid_lookup(
        jnp.reshape(x, (N * DIM,)),
        jnp.reshape(table_lin, (HASHMAP, NFEAT)),
    )
    return jnp.reshape(out_flat, (N, NFEAT))


# speedup vs baseline: 125.9671x; 2.0954x over previous
import functools

import jax
import jax.numpy as jnp
import numpy as np
from jax import lax
from jax.experimental import pallas as pl
from jax.experimental.pallas import tpu as pltpu
from jax.experimental.pallas import tpu_sc as plsc

DIM = 3
NFEAT = 8
HASHMAP = 524288
RES = 128.0
N = 524288

NC, NS, L = 2, 16, 16
NW = NC * NS
PPW = N // NW
C = 512
NG = C // L
NCHUNK = PPW // C

P1 = np.int32(np.uint32(2654435761).astype(np.int32))
P2 = np.int32(805459861)
HMASK = np.int32(HASHMAP - 1)


def kernel(x, table):
    mesh = plsc.VectorSubcoreMesh(core_axis_name="c", subcore_axis_name="s")

    @functools.partial(
        pl.kernel,
        mesh=mesh,
        out_type=jax.ShapeDtypeStruct((N * NFEAT,), jnp.float32),
        scratch_types=[
            pltpu.VMEM((2, DIM, C), jnp.float32),
            pltpu.VMEM((2 * NG, 8 * L), jnp.int32),
            pltpu.VMEM((2, C * 8, NFEAT), jnp.float32),
            pltpu.VMEM((C * NFEAT,), jnp.float32),
            pltpu.SemaphoreType.DMA,
            pltpu.SemaphoreType.DMA,
        ],
        compiler_params=pltpu.CompilerParams(
            needs_layout_passes=False, use_tc_tiling_on_sc=False
        ),
    )
    def grid_lookup(x_hbm, table_hbm, out_hbm, xbuf, idxbuf, rowsbuf, outbuf,
                    sem0, sem1):
        i32 = jnp.int32
        wid = (lax.axis_index("s").astype(i32) * i32(NC)
               + lax.axis_index("c").astype(i32))
        iota = lax.iota(jnp.int32, L)
        sems = (sem0, sem1)
        half = iota // i32(8)
        pidx = [jnp.full((L,), 2 * k, jnp.int32) + half for k in range(8)]
        fcol2 = iota % i32(8)

        _dn = lax.GatherDimensionNumbers(
            offset_dims=(), collapsed_slice_dims=(0,), start_index_map=(0,)
        )

        def vgather(v, idx16):
            return lax.gather(
                v, idx16[:, None], _dn, (1,),
                mode=lax.GatherScatterMode.PROMISE_IN_BOUNDS,
            )

        def load_xs(p, g):
            g16 = g * i32(L)
            xs0 = xbuf[np.int32(p), np.int32(0), pl.ds(g16, L)] * RES
            xs1 = xbuf[np.int32(p), np.int32(1), pl.ds(g16, L)] * RES
            xs2 = xbuf[np.int32(p), np.int32(2), pl.ds(g16, L)] * RES
            return xs0, xs1, xs2

        def make_hash_group(p):
            def hash_group(g, _):
                xs0, xs1, xs2 = load_xs(p, g)
                xi0 = xs0.astype(jnp.int32)
                xi1 = xs1.astype(jnp.int32)
                xi2 = xs2.astype(jnp.int32)
                h0, hp0 = xi0, xi0 + i32(1)
                h1 = xi1 * P1
                hp1 = h1 + P1
                h2 = xi2 * P2
                hp2 = h2 + P2
                for c in range(8):
                    a = hp0 if c & 1 else h0
                    b = hp1 if c & 2 else h1
                    d = hp2 if c & 4 else h2
                    idxbuf[i32(p * NG) + g, pl.ds(c * L, L)] = (a ^ b ^ d) & HMASK
                return i32(0)

            return hash_group

        hash_groups = [make_hash_group(0), make_hash_group(1)]

        def make_compute_group(p):
            def compute_group(g, _):
                xs0, xs1, xs2 = load_xs(p, g)
                rb = rowsbuf.at[np.int32(p)]
                xf0 = xs0 - xs0.astype(jnp.int32).astype(jnp.float32)
                xf1 = xs1 - xs1.astype(jnp.int32).astype(jnp.float32)
                xf2 = xs2 - xs2.astype(jnp.int32).astype(jnp.float32)
                m0, m1, m2 = 1.0 - xf0, 1.0 - xf1, 1.0 - xf2
                ws = []
                for c in range(8):
                    t0 = xf0 if c & 1 else m0
                    t1 = xf1 if c & 2 else m1
                    t2 = xf2 if c & 4 else m2
                    ws.append(t0 * t1 * t2)
                acc = [None] * 8
                for c in range(8):
                    rbase = g * i32(8 * L) + i32(c * L)
                    wc = ws[c]
                    for k in range(8):
                        v = plsc.load_gather(rb, [rbase + pidx[k], fcol2])
                        wp = vgather(wc, pidx[k])
                        acc[k] = wp * v if c == 0 else acc[k] + wp * v
                gout = g * i32(L * NFEAT)
                for k in range(8):
                    outbuf[pl.ds(gout + i32(k * L), L)] = acc[k]
                return i32(0)

            return compute_group

        compute_groups = [make_compute_group(0), make_compute_group(1)]

        def cbase_of(k):
            return wid * i32(PPW) + k * i32(C)

        def prepare(p, k):
            pltpu.sync_copy(x_hbm.at[:, pl.ds(cbase_of(k), C)],
                            xbuf.at[np.int32(p)])
            lax.fori_loop(i32(0), i32(NG), hash_groups[p], i32(0))
            for g in range(NG):
                pltpu.async_copy(
                    table_hbm.at[idxbuf.at[np.int32(p * NG + g)]],
                    rowsbuf.at[np.int32(p), pl.ds(g * (8 * L), 8 * L)],
                    sems[p],
                )

        def finish(p, k):
            for g in range(NG):
                pltpu.make_async_copy(
                    table_hbm.at[idxbuf.at[np.int32(p * NG + g)]],
                    rowsbuf.at[np.int32(p), pl.ds(g * (8 * L), 8 * L)],
                    sems[p],
                ).wait()
            lax.fori_loop(i32(0), i32(NG), compute_groups[p], i32(0))
            pltpu.sync_copy(outbuf,
                            out_hbm.at[pl.ds(cbase_of(k) * i32(NFEAT), C * NFEAT)])

        prepare(0, i32(0))

        def pair_body(k2, _):
            k0 = k2 * i32(2)
            prepare(1, k0 + i32(1))
            finish(0, k0)

            @pl.when(k0 + i32(2) < i32(NCHUNK))
            def _():
                prepare(0, k0 + i32(2))

            finish(1, k0 + i32(1))
            return i32(0)

        lax.fori_loop(i32(0), i32(NCHUNK // 2), pair_body, i32(0))

    out_flat = grid_lookup(jnp.swapaxes(x, 0, 1), table)
    return jnp.reshape(out_flat, (N, NFEAT))


# speedup vs baseline: 203.3254x; 1.6141x over previous
import functools

import jax
import jax.numpy as jnp
import numpy as np
from jax import lax
from jax.experimental import pallas as pl
from jax.experimental.pallas import tpu as pltpu
from jax.experimental.pallas import tpu_sc as plsc

DIM = 3
NFEAT = 8
HASHMAP = 524288
RES = 128.0
N = 524288

NC, NS, L = 2, 16, 16
NW = NC * NS
PPW = N // NW
C = 512
NG = C // L
NCHUNK = PPW // C

P1 = np.int32(np.uint32(2654435761).astype(np.int32))
P2 = np.int32(805459861)
HMASK = np.int32(HASHMAP - 1)


def kernel(x, table):
    mesh = plsc.VectorSubcoreMesh(core_axis_name="c", subcore_axis_name="s")

    @functools.partial(
        pl.kernel,
        mesh=mesh,
        out_type=jax.ShapeDtypeStruct((N // 128, NFEAT, 128), jnp.float32),
        scratch_types=[
            pltpu.VMEM((2, DIM, C), jnp.float32),
            pltpu.VMEM((2 * NG, 8 * L), jnp.int32),
            pltpu.VMEM((2, C * 8, NFEAT), jnp.float32),
            pltpu.VMEM((C // 128, NFEAT, 130), jnp.float32),
            pltpu.SemaphoreType.DMA,
            pltpu.SemaphoreType.DMA,
        ],
        compiler_params=pltpu.CompilerParams(
            needs_layout_passes=False, use_tc_tiling_on_sc=False
        ),
    )
    def grid_lookup(x_hbm, table_hbm, out_hbm, xbuf, idxbuf, rowsbuf, outbuf,
                    sem0, sem1):
        i32 = jnp.int32
        wid = (lax.axis_index("s").astype(i32) * i32(NC)
               + lax.axis_index("c").astype(i32))
        iota = lax.iota(jnp.int32, L)
        sems = (sem0, sem1)
        half = iota // i32(8)
        pidx = [jnp.full((L,), 2 * k, jnp.int32) + half for k in range(8)]
        fcol2 = iota % i32(8)
        zero16 = jnp.zeros((L,), jnp.int32)

        _dn = lax.GatherDimensionNumbers(
            offset_dims=(), collapsed_slice_dims=(0,), start_index_map=(0,)
        )

        def vgather(v, idx16):
            return lax.gather(
                v, idx16[:, None], _dn, (1,),
                mode=lax.GatherScatterMode.PROMISE_IN_BOUNDS,
            )

        def load_xs(p, g):
            g16 = g * i32(L)
            xs0 = xbuf[np.int32(p), np.int32(0), pl.ds(g16, L)] * RES
            xs1 = xbuf[np.int32(p), np.int32(1), pl.ds(g16, L)] * RES
            xs2 = xbuf[np.int32(p), np.int32(2), pl.ds(g16, L)] * RES
            return xs0, xs1, xs2

        def make_hash_group(p):
            def hash_group(g, _):
                xs0, xs1, xs2 = load_xs(p, g)
                xi0 = xs0.astype(jnp.int32)
                xi1 = xs1.astype(jnp.int32)
                xi2 = xs2.astype(jnp.int32)
                h0, hp0 = xi0, xi0 + i32(1)
                h1 = xi1 * P1
                hp1 = h1 + P1
                h2 = xi2 * P2
                hp2 = h2 + P2
                for c in range(8):
                    a = hp0 if c & 1 else h0
                    b = hp1 if c & 2 else h1
                    d = hp2 if c & 4 else h2
                    idxbuf[i32(p * NG) + g, pl.ds(c * L, L)] = (a ^ b ^ d) & HMASK
                return i32(0)

            return hash_group

        hash_groups = [make_hash_group(0), make_hash_group(1)]

        def make_compute_group(p):
            def compute_group(g, _):
                xs0, xs1, xs2 = load_xs(p, g)
                rb = rowsbuf.at[np.int32(p)]
                xf0 = xs0 - xs0.astype(jnp.int32).astype(jnp.float32)
                xf1 = xs1 - xs1.astype(jnp.int32).astype(jnp.float32)
                xf2 = xs2 - xs2.astype(jnp.int32).astype(jnp.float32)
                m0, m1, m2 = 1.0 - xf0, 1.0 - xf1, 1.0 - xf2
                ws = []
                for c in range(8):
                    t0 = xf0 if c & 1 else m0
                    t1 = xf1 if c & 2 else m1
                    t2 = xf2 if c & 4 else m2
                    ws.append(t0 * t1 * t2)
                acc = [None] * 8
                for c in range(8):
                    rbase = g * i32(8 * L) + i32(c * L)
                    wc = ws[c]
                    for k in range(8):
                        v = plsc.load_gather(rb, [rbase + pidx[k], fcol2])
                        wp = vgather(wc, pidx[k])
                        acc[k] = wp * v if c == 0 else acc[k] + wp * v
                blk = g // i32(8)
                inner0 = (g % i32(8)) * i32(L)
                bvec = zero16 + blk
                for k in range(8):
                    plsc.store_scatter(
                        outbuf,
                        [bvec, fcol2, inner0 + i32(2 * k) + half],
                        acc[k],
                    )
                return i32(0)

            return compute_group

        compute_groups = [make_compute_group(0), make_compute_group(1)]

        def cbase_of(k):
            return wid * i32(PPW) + k * i32(C)

        def prepare(p, k):
            pltpu.sync_copy(x_hbm.at[:, pl.ds(cbase_of(k), C)],
                            xbuf.at[np.int32(p)])
            lax.fori_loop(i32(0), i32(NG), hash_groups[p], i32(0))
            for g in range(NG):
                pltpu.async_copy(
                    table_hbm.at[idxbuf.at[np.int32(p * NG + g)]],
                    rowsbuf.at[np.int32(p), pl.ds(g * (8 * L), 8 * L)],
                    sems[p],
                )

        def finish(p, k):
            for g in range(NG):
                pltpu.make_async_copy(
                    table_hbm.at[idxbuf.at[np.int32(p * NG + g)]],
                    rowsbuf.at[np.int32(p), pl.ds(g * (8 * L), 8 * L)],
                    sems[p],
                ).wait()
            lax.fori_loop(i32(0), i32(NG), compute_groups[p], i32(0))
            pltpu.sync_copy(outbuf.at[:, :, pl.ds(0, 128)],
                            out_hbm.at[pl.ds(cbase_of(k) // i32(128), C // 128)])

        prepare(0, i32(0))

        def pair_body(k2, _):
            k0 = k2 * i32(2)
            prepare(1, k0 + i32(1))
            finish(0, k0)

            @pl.when(k0 + i32(2) < i32(NCHUNK))
            def _():
                prepare(0, k0 + i32(2))

            finish(1, k0 + i32(1))
            return i32(0)

        lax.fori_loop(i32(0), i32(NCHUNK // 2), pair_body, i32(0))

    t128 = lax.optimization_barrier(jnp.reshape(table, (HASHMAP * NFEAT // 128, 128)))
    out3d = grid_lookup(
        jnp.swapaxes(x, 0, 1),
        jnp.reshape(t128, (HASHMAP, NFEAT)),
    )
    return jnp.reshape(jnp.transpose(out3d, (0, 2, 1)), (N, NFEAT))


# speedup vs baseline: 324.2023x; 1.5945x over previous
import functools

import jax
import jax.numpy as jnp
import numpy as np
from jax import lax
from jax.experimental import pallas as pl
from jax.experimental.pallas import tpu as pltpu
from jax.experimental.pallas import tpu_sc as plsc

DIM = 3
NFEAT = 8
HASHMAP = 524288
RES = 128.0
N = 524288

NC, NS, L = 2, 16, 16
NW = NC * NS
PPW = N // NW
C = 512
NG = C // L
NCHUNK = PPW // C

P1 = np.int32(np.uint32(2654435761).astype(np.int32))
P2 = np.int32(805459861)
HMASK = np.int32(HASHMAP - 1)


NTILES = HASHMAP // 128
TPW = NTILES // NW
TB = 32


def _transpose_table(tv):
    mesh = plsc.VectorSubcoreMesh(core_axis_name="c", subcore_axis_name="s")

    @functools.partial(
        pl.kernel,
        mesh=mesh,
        out_type=jax.ShapeDtypeStruct((HASHMAP, NFEAT), jnp.float32),
        scratch_types=[
            pltpu.VMEM((TB, NFEAT, 130), jnp.float32),
            pltpu.VMEM((TB * 128, NFEAT), jnp.float32),
        ],
        compiler_params=pltpu.CompilerParams(
            needs_layout_passes=False, use_tc_tiling_on_sc=False
        ),
    )
    def transpose_k(tv_hbm, trm_hbm, inbuf, outbuf):
        i32 = jnp.int32
        wid = (lax.axis_index("s").astype(i32) * i32(NC)
               + lax.axis_index("c").astype(i32))
        iota = lax.iota(jnp.int32, 16)
        half = iota // i32(8)
        fcol2 = iota % i32(8)
        zero16 = jnp.zeros((16,), jnp.int32)

        def batch(b, _):
            tb = wid * i32(TPW) + b * i32(TB)
            pltpu.sync_copy(tv_hbm.at[pl.ds(tb, TB)],
                            inbuf.at[:, :, pl.ds(0, 128)])

            def tile(tt, _):
                ttvec = zero16 + tt
                for j0 in range(64):
                    v = plsc.load_gather(
                        inbuf, [ttvec, fcol2, i32(2 * j0) + half])
                    rowvec = tt * i32(128) + i32(2 * j0) + half
                    plsc.store_scatter(outbuf, [rowvec, fcol2], v)
                return i32(0)

            lax.fori_loop(i32(0), i32(TB), tile, i32(0))
            pltpu.sync_copy(outbuf, trm_hbm.at[pl.ds(tb * i32(128), TB * 128)])
            return i32(0)

        lax.fori_loop(i32(0), i32(TPW // TB), batch, i32(0))

    return transpose_k(tv)


def kernel(x, table):
    mesh = plsc.VectorSubcoreMesh(core_axis_name="c", subcore_axis_name="s")

    @functools.partial(
        pl.kernel,
        mesh=mesh,
        out_type=jax.ShapeDtypeStruct((N // 128, NFEAT, 128), jnp.float32),
        scratch_types=[
            pltpu.VMEM((2, DIM, C), jnp.float32),
            pltpu.VMEM((2 * NG, 8 * L), jnp.int32),
            pltpu.VMEM((2, C * 8, NFEAT), jnp.float32),
            pltpu.VMEM((C // 128, NFEAT, 130), jnp.float32),
            pltpu.SemaphoreType.DMA,
            pltpu.SemaphoreType.DMA,
        ],
        compiler_params=pltpu.CompilerParams(
            needs_layout_passes=False, use_tc_tiling_on_sc=False
        ),
    )
    def grid_lookup(x_hbm, table_hbm, out_hbm, xbuf, idxbuf, rowsbuf, outbuf,
                    sem0, sem1):
        i32 = jnp.int32
        wid = (lax.axis_index("s").astype(i32) * i32(NC)
               + lax.axis_index("c").astype(i32))
        iota = lax.iota(jnp.int32, L)
        sems = (sem0, sem1)
        half = iota // i32(8)
        pidx = [jnp.full((L,), 2 * k, jnp.int32) + half for k in range(8)]
        fcol2 = iota % i32(8)
        zero16 = jnp.zeros((L,), jnp.int32)

        _dn = lax.GatherDimensionNumbers(
            offset_dims=(), collapsed_slice_dims=(0,), start_index_map=(0,)
        )

        def vgather(v, idx16):
            return lax.gather(
                v, idx16[:, None], _dn, (1,),
                mode=lax.GatherScatterMode.PROMISE_IN_BOUNDS,
            )

        def load_xs(p, g):
            g16 = g * i32(L)
            xs0 = xbuf[np.int32(p), np.int32(0), pl.ds(g16, L)] * RES
            xs1 = xbuf[np.int32(p), np.int32(1), pl.ds(g16, L)] * RES
            xs2 = xbuf[np.int32(p), np.int32(2), pl.ds(g16, L)] * RES
            return xs0, xs1, xs2

        def make_hash_group(p):
            def hash_group(g, _):
                xs0, xs1, xs2 = load_xs(p, g)
                xi0 = xs0.astype(jnp.int32)
                xi1 = xs1.astype(jnp.int32)
                xi2 = xs2.astype(jnp.int32)
                h0, hp0 = xi0, xi0 + i32(1)
                h1 = xi1 * P1
                hp1 = h1 + P1
                h2 = xi2 * P2
                hp2 = h2 + P2
                for c in range(8):
                    a = hp0 if c & 1 else h0
                    b = hp1 if c & 2 else h1
                    d = hp2 if c & 4 else h2
                    idxbuf[i32(p * NG) + g, pl.ds(c * L, L)] = (a ^ b ^ d) & HMASK
                return i32(0)

            return hash_group

        hash_groups = [make_hash_group(0), make_hash_group(1)]

        def make_compute_group(p):
            def compute_group(g, _):
                xs0, xs1, xs2 = load_xs(p, g)
                rb = rowsbuf.at[np.int32(p)]
                xf0 = xs0 - xs0.astype(jnp.int32).astype(jnp.float32)
                xf1 = xs1 - xs1.astype(jnp.int32).astype(jnp.float32)
                xf2 = xs2 - xs2.astype(jnp.int32).astype(jnp.float32)
                m0, m1, m2 = 1.0 - xf0, 1.0 - xf1, 1.0 - xf2
                ws = []
                for c in range(8):
                    t0 = xf0 if c & 1 else m0
                    t1 = xf1 if c & 2 else m1
                    t2 = xf2 if c & 4 else m2
                    ws.append(t0 * t1 * t2)
                acc = [None] * 8
                for c in range(8):
                    rbase = g * i32(8 * L) + i32(c * L)
                    wc = ws[c]
                    for k in range(8):
                        v = plsc.load_gather(rb, [rbase + pidx[k], fcol2])
                        wp = vgather(wc, pidx[k])
                        acc[k] = wp * v if c == 0 else acc[k] + wp * v
                blk = g // i32(8)
                inner0 = (g % i32(8)) * i32(L)
                bvec = zero16 + blk
                for k in range(8):
                    plsc.store_scatter(
                        outbuf,
                        [bvec, fcol2, inner0 + i32(2 * k) + half],
                        acc[k],
                    )
                return i32(0)

            return compute_group

        compute_groups = [make_compute_group(0), make_compute_group(1)]

        def cbase_of(k):
            return wid * i32(PPW) + k * i32(C)

        def prepare(p, k):
            pltpu.sync_copy(x_hbm.at[:, pl.ds(cbase_of(k), C)],
                            xbuf.at[np.int32(p)])
            lax.fori_loop(i32(0), i32(NG), hash_groups[p], i32(0))
            for g in range(NG):
                pltpu.async_copy(
                    table_hbm.at[idxbuf.at[np.int32(p * NG + g)]],
                    rowsbuf.at[np.int32(p), pl.ds(g * (8 * L), 8 * L)],
                    sems[p],
                )

        def finish(p, k):
            for g in range(NG):
                pltpu.make_async_copy(
                    table_hbm.at[idxbuf.at[np.int32(p * NG + g)]],
                    rowsbuf.at[np.int32(p), pl.ds(g * (8 * L), 8 * L)],
                    sems[p],
                ).wait()
            lax.fori_loop(i32(0), i32(NG), compute_groups[p], i32(0))
            pltpu.sync_copy(outbuf.at[:, :, pl.ds(0, 128)],
                            out_hbm.at[pl.ds(cbase_of(k) // i32(128), C // 128)])

        prepare(0, i32(0))

        def pair_body(k2, _):
            k0 = k2 * i32(2)
            prepare(1, k0 + i32(1))
            finish(0, k0)

            @pl.when(k0 + i32(2) < i32(NCHUNK))
            def _():
                prepare(0, k0 + i32(2))

            finish(1, k0 + i32(1))
            return i32(0)

        lax.fori_loop(i32(0), i32(NCHUNK // 2), pair_body, i32(0))

    tv = jnp.transpose(jnp.reshape(table, (NTILES, 128, NFEAT)), (0, 2, 1))
    trm = _transpose_table(tv)
    out3d = grid_lookup(jnp.swapaxes(x, 0, 1), trm)
    return jnp.reshape(jnp.transpose(out3d, (0, 2, 1)), (N, NFEAT))


# speedup vs baseline: 324.7873x; 1.0018x over previous
import functools

import jax
import jax.numpy as jnp
import numpy as np
from jax import lax
from jax.experimental import pallas as pl
from jax.experimental.pallas import tpu as pltpu
from jax.experimental.pallas import tpu_sc as plsc

DIM = 3
NFEAT = 8
HASHMAP = 524288
RES = 128.0
N = 524288

NC, NS, L = 2, 16, 16
NW = NC * NS
PPW = N // NW
C = 512
NG = C // L
NCHUNK = PPW // C

P1 = np.int32(np.uint32(2654435761).astype(np.int32))
P2 = np.int32(805459861)
HMASK = np.int32(HASHMAP - 1)


NTILES = HASHMAP // 128
TPW = NTILES // NW
TB = 16
NB = TPW // TB


def _transpose_table(tv):
    mesh = plsc.VectorSubcoreMesh(core_axis_name="c", subcore_axis_name="s")

    @functools.partial(
        pl.kernel,
        mesh=mesh,
        out_type=jax.ShapeDtypeStruct((HASHMAP, NFEAT), jnp.float32),
        scratch_types=[
            pltpu.VMEM((2 * TB, NFEAT, 130), jnp.float32),
            pltpu.VMEM((2 * TB * 128, NFEAT), jnp.float32),
            pltpu.SemaphoreType.DMA,
            pltpu.SemaphoreType.DMA,
            pltpu.SemaphoreType.DMA,
            pltpu.SemaphoreType.DMA,
        ],
        compiler_params=pltpu.CompilerParams(
            needs_layout_passes=False, use_tc_tiling_on_sc=False
        ),
    )
    def transpose_k(tv_hbm, trm_hbm, inbuf, outbuf, si0, si1, so0, so1):
        i32 = jnp.int32
        wid = (lax.axis_index("s").astype(i32) * i32(NC)
               + lax.axis_index("c").astype(i32))
        iota = lax.iota(jnp.int32, 16)
        half = iota // i32(8)
        fcol2 = iota % i32(8)
        zero16 = jnp.zeros((16,), jnp.int32)
        sins = (si0, si1)
        souts = (so0, so1)

        def tbase(b):
            return wid * i32(TPW) + b * i32(TB)

        def in_copy(p, b):
            return pltpu.make_async_copy(
                tv_hbm.at[pl.ds(tbase(b), TB)],
                inbuf.at[pl.ds(p * TB, TB), :, pl.ds(0, 128)],
                sins[p],
            )

        def out_copy(p, b):
            return pltpu.make_async_copy(
                outbuf.at[pl.ds(p * TB * 128, TB * 128)],
                trm_hbm.at[pl.ds(tbase(b) * i32(128), TB * 128)],
                souts[p],
            )

        def compute(p):
            def tile(tt, _):
                ttvec = zero16 + tt + i32(p * TB)
                for j0 in range(64):
                    v = plsc.load_gather(
                        inbuf, [ttvec, fcol2, i32(2 * j0) + half])
                    rowvec = (tt + i32(p * TB)) * i32(128) + i32(2 * j0) + half
                    plsc.store_scatter(outbuf, [rowvec, fcol2], v)
                return i32(0)

            lax.fori_loop(i32(0), i32(TB), tile, i32(0))

        in_copy(0, i32(0)).start()

        def pair_body(b2, _):
            b0 = b2 * i32(2)
            in_copy(1, b0 + i32(1)).start()
            in_copy(0, b0).wait()

            @pl.when(b2 >= i32(1))
            def _():
                out_copy(0, b0 - i32(2)).wait()

            compute(0)
            out_copy(0, b0).start()

            @pl.when(b0 + i32(2) < i32(NB))
            def _():
                in_copy(0, b0 + i32(2)).start()

            in_copy(1, b0 + i32(1)).wait()

            @pl.when(b2 >= i32(1))
            def _():
                out_copy(1, b0 - i32(1)).wait()

            compute(1)
            out_copy(1, b0 + i32(1)).start()
            return i32(0)

        lax.fori_loop(i32(0), i32(NB // 2), pair_body, i32(0))
        out_copy(0, i32(NB - 2)).wait()
        out_copy(1, i32(NB - 1)).wait()

    return transpose_k(tv)


def kernel(x, table):
    mesh = plsc.VectorSubcoreMesh(core_axis_name="c", subcore_axis_name="s")

    @functools.partial(
        pl.kernel,
        mesh=mesh,
        out_type=jax.ShapeDtypeStruct((N // 128, NFEAT, 128), jnp.float32),
        scratch_types=[
            pltpu.VMEM((2, DIM, C), jnp.float32),
            pltpu.VMEM((2 * NG, 8 * L), jnp.int32),
            pltpu.VMEM((2, C * 8, NFEAT), jnp.float32),
            pltpu.VMEM((C // 128, NFEAT, 130), jnp.float32),
            pltpu.SemaphoreType.DMA,
            pltpu.SemaphoreType.DMA,
        ],
        compiler_params=pltpu.CompilerParams(
            needs_layout_passes=False, use_tc_tiling_on_sc=False
        ),
    )
    def grid_lookup(x_hbm, table_hbm, out_hbm, xbuf, idxbuf, rowsbuf, outbuf,
                    sem0, sem1):
        i32 = jnp.int32
        wid = (lax.axis_index("s").astype(i32) * i32(NC)
               + lax.axis_index("c").astype(i32))
        iota = lax.iota(jnp.int32, L)
        sems = (sem0, sem1)
        half = iota // i32(8)
        pidx = [jnp.full((L,), 2 * k, jnp.int32) + half for k in range(8)]
        fcol2 = iota % i32(8)
        zero16 = jnp.zeros((L,), jnp.int32)

        _dn = lax.GatherDimensionNumbers(
            offset_dims=(), collapsed_slice_dims=(0,), start_index_map=(0,)
        )

        def vgather(v, idx16):
            return lax.gather(
                v, idx16[:, None], _dn, (1,),
                mode=lax.GatherScatterMode.PROMISE_IN_BOUNDS,
            )

        def load_xs(p, g):
            g16 = g * i32(L)
            xs0 = xbuf[np.int32(p), np.int32(0), pl.ds(g16, L)] * RES
            xs1 = xbuf[np.int32(p), np.int32(1), pl.ds(g16, L)] * RES
            xs2 = xbuf[np.int32(p), np.int32(2), pl.ds(g16, L)] * RES
            return xs0, xs1, xs2

        def make_hash_group(p):
            def hash_group(g, _):
                xs0, xs1, xs2 = load_xs(p, g)
                xi0 = xs0.astype(jnp.int32)
                xi1 = xs1.astype(jnp.int32)
                xi2 = xs2.astype(jnp.int32)
                h0, hp0 = xi0, xi0 + i32(1)
                h1 = xi1 * P1
                hp1 = h1 + P1
                h2 = xi2 * P2
                hp2 = h2 + P2
                for c in range(8):
                    a = hp0 if c & 1 else h0
                    b = hp1 if c & 2 else h1
                    d = hp2 if c & 4 else h2
                    idxbuf[i32(p * NG) + g, pl.ds(c * L, L)] = (a ^ b ^ d) & HMASK
                return i32(0)

            return hash_group

        hash_groups = [make_hash_group(0), make_hash_group(1)]

        def make_compute_group(p):
            def compute_group(g, _):
                xs0, xs1, xs2 = load_xs(p, g)
                rb = rowsbuf.at[np.int32(p)]
                xf0 = xs0 - xs0.astype(jnp.int32).astype(jnp.float32)
                xf1 = xs1 - xs1.astype(jnp.int32).astype(jnp.float32)
                xf2 = xs2 - xs2.astype(jnp.int32).astype(jnp.float32)
                m0, m1, m2 = 1.0 - xf0, 1.0 - xf1, 1.0 - xf2
                ws = []
                for c in range(8):
                    t0 = xf0 if c & 1 else m0
                    t1 = xf1 if c & 2 else m1
                    t2 = xf2 if c & 4 else m2
                    ws.append(t0 * t1 * t2)
                acc = [None] * 8
                for c in range(8):
                    rbase = g * i32(8 * L) + i32(c * L)
                    wc = ws[c]
                    for k in range(8):
                        v = plsc.load_gather(rb, [rbase + pidx[k], fcol2])
                        wp = vgather(wc, pidx[k])
                        acc[k] = wp * v if c == 0 else acc[k] + wp * v
                blk = g // i32(8)
                inner0 = (g % i32(8)) * i32(L)
                bvec = zero16 + blk
                for k in range(8):
                    plsc.store_scatter(
                        outbuf,
                        [bvec, fcol2, inner0 + i32(2 * k) + half],
                        acc[k],
                    )
                return i32(0)

            return compute_group

        compute_groups = [make_compute_group(0), make_compute_group(1)]

        def cbase_of(k):
            return wid * i32(PPW) + k * i32(C)

        def prepare(p, k):
            pltpu.sync_copy(x_hbm.at[:, pl.ds(cbase_of(k), C)],
                            xbuf.at[np.int32(p)])
            lax.fori_loop(i32(0), i32(NG), hash_groups[p], i32(0))
            for g in range(NG):
                pltpu.async_copy(
                    table_hbm.at[idxbuf.at[np.int32(p * NG + g)]],
                    rowsbuf.at[np.int32(p), pl.ds(g * (8 * L), 8 * L)],
                    sems[p],
                )

        def finish(p, k):
            for g in range(NG):
                pltpu.make_async_copy(
                    table_hbm.at[idxbuf.at[np.int32(p * NG + g)]],
                    rowsbuf.at[np.int32(p), pl.ds(g * (8 * L), 8 * L)],
                    sems[p],
                ).wait()
            lax.fori_loop(i32(0), i32(NG), compute_groups[p], i32(0))
            pltpu.sync_copy(outbuf.at[:, :, pl.ds(0, 128)],
                            out_hbm.at[pl.ds(cbase_of(k) // i32(128), C // 128)])

        prepare(0, i32(0))

        def pair_body(k2, _):
            k0 = k2 * i32(2)
            prepare(1, k0 + i32(1))
            finish(0, k0)

            @pl.when(k0 + i32(2) < i32(NCHUNK))
            def _():
                prepare(0, k0 + i32(2))

            finish(1, k0 + i32(1))
            return i32(0)

        lax.fori_loop(i32(0), i32(NCHUNK // 2), pair_body, i32(0))

    tv = jnp.transpose(jnp.reshape(table, (NTILES, 128, NFEAT)), (0, 2, 1))
    trm = _transpose_table(tv)
    out3d = grid_lookup(jnp.swapaxes(x, 0, 1), trm)
    return jnp.reshape(jnp.transpose(out3d, (0, 2, 1)), (N, NFEAT))
